# Initial kernel scaffold; baseline (speedup 1.0000x reference)
import jax
import jax.numpy as jnp
from jax.experimental import pallas as pl


def kernel(syn_bits, r_list, mask, det_emb_w, rnd_emb_w, proj_w, alpha):
    raise NotImplementedError("write your pallas kernel here")


# speedup vs baseline: 1.2693x; 1.2693x over previous
import functools

import jax
import jax.numpy as jnp
from jax import lax
from jax.experimental import pallas as pl
from jax.experimental.pallas import tpu as pltpu
from jax.experimental.pallas import tpu_sc as plsc

_NUM_DETECTORS = 4096
_MAX_ROUNDS = 64
_DIM = 256
_B = 16
_SYN_LEN = 4096
_RND_PAD = 128

_NC = 2
_NS = 16
_NW = _NC * _NS
_L = 16
_CHUNK = _SYN_LEN // _NW
_G = _CHUNK // _L


def _dots_tc(det_ref, rnd_ref, proj_ref, alpha_ref, adet_ref, arnd_ref):
    a = alpha_ref[0, 0]
    proj = proj_ref[...]
    dn = (((1,), (1,)), ((), ()))
    adet = lax.dot_general(proj, det_ref[...], dn,
                           preferred_element_type=jnp.float32)
    arnd = lax.dot_general(proj, rnd_ref[...], dn,
                           preferred_element_type=jnp.float32)
    adet_ref[...] = a * adet
    arnd_ref[...] = a * arnd


def _sc_body(syn_hbm, mask_hbm, rlist_hbm, adet_hbm, arnd_hbm, out_hbm,
             adet_v, arnd_v, r_v, syn_v, mask_v, out_v, pe_v):
    wid = lax.axis_index("s") * _NC + lax.axis_index("c")
    base = wid * _CHUNK

    pltpu.sync_copy(adet_hbm, adet_v)
    pltpu.sync_copy(arnd_hbm, arnd_v)
    pltpu.sync_copy(rlist_hbm, r_v)
    pltpu.sync_copy(syn_hbm.at[:, pl.ds(base, _CHUNK)], syn_v)
    pltpu.sync_copy(mask_hbm.at[:, pl.ds(base, _CHUNK)], mask_v)

    r = r_v[...]
    d = lax.div(jnp.full((_L,), _SYN_LEN, jnp.int32), r)
    for g in range(_G):
        p = lax.broadcasted_iota(jnp.int32, (_L,), 0) + (base + g * _L)
        q = lax.div(p, d)
        det_id = p - q * d
        rnd_id = jnp.minimum(q + 1, _MAX_ROUNDS)
        pe = (plsc.load_gather(adet_v, [det_id]) +
              plsc.load_gather(arnd_v, [rnd_id]))
        pe_v[pl.ds(g * _L, _L)] = pe

    for b in range(_B):
        for g in range(_G):
            sl = pl.ds(g * _L, _L)
            out_v[b, sl] = syn_v[b, sl] + mask_v[b, sl] * pe_v[sl]

    pltpu.sync_copy(out_v, out_hbm.at[:, pl.ds(base, _CHUNK)])


@jax.jit
def kernel(syn_bits, r_list, mask, det_emb_w, rnd_emb_w, proj_w, alpha):
    rnd_pad = jnp.zeros((_RND_PAD, _DIM), jnp.float32).at[:_MAX_ROUNDS + 1].set(
        rnd_emb_w)
    alpha2d = jnp.reshape(alpha, (1, 1)).astype(jnp.float32)

    adet, arnd = pl.pallas_call(
        _dots_tc,
        out_shape=(
            jax.ShapeDtypeStruct((1, _NUM_DETECTORS), jnp.float32),
            jax.ShapeDtypeStruct((1, _RND_PAD), jnp.float32),
        ),
    )(det_emb_w, rnd_pad, proj_w, alpha2d)
    adet = jnp.reshape(adet, (_NUM_DETECTORS,))
    arnd = jnp.reshape(arnd, (_RND_PAD,))

    mesh = plsc.VectorSubcoreMesh(core_axis_name="c", subcore_axis_name="s",
                                  num_cores=_NC, num_subcores=_NS)
    sc = pl.kernel(
        _sc_body,
        out_type=jax.ShapeDtypeStruct((_B, _SYN_LEN), jnp.float32),
        mesh=mesh,
        compiler_params=pltpu.CompilerParams(needs_layout_passes=False),
        scratch_types=[
            pltpu.VMEM((_NUM_DETECTORS,), jnp.float32),
            pltpu.VMEM((_RND_PAD,), jnp.float32),
            pltpu.VMEM((_L,), jnp.int32),
            pltpu.VMEM((_B, _CHUNK), jnp.float32),
            pltpu.VMEM((_B, _CHUNK), jnp.float32),
            pltpu.VMEM((_B, _CHUNK), jnp.float32),
            pltpu.VMEM((_CHUNK,), jnp.float32),
        ],
    )
    return sc(syn_bits, mask, r_list, adet, arnd)


# speedup vs baseline: 1.5375x; 1.2113x over previous
import jax
import jax.numpy as jnp
from jax import lax
from jax.experimental import pallas as pl
from jax.experimental.pallas import tpu as pltpu
from jax.experimental.pallas import tpu_sc as plsc

_NUM_DETECTORS = 4096
_MAX_ROUNDS = 64
_DIM = 256
_B = 16
_SYN_LEN = 4096
_NRND = _MAX_ROUNDS + 1

_NC = 2
_NS = 16
_NW = _NC * _NS
_L = 16
_CHUNK = _SYN_LEN // _NW
_G = _CHUNK // _L


def _dots_tc(det_ref, rnd_ref, proj_ref, alpha_ref, adet_ref, arnd_ref):
    a = alpha_ref[0, 0]
    proj = proj_ref[...]
    dn = (((1,), (1,)), ((), ()))
    adet = lax.dot_general(proj, det_ref[...], dn,
                           preferred_element_type=jnp.float32)
    arnd = lax.dot_general(proj, rnd_ref[...], dn,
                           preferred_element_type=jnp.float32)
    adet_ref[...] = a * adet
    arnd_ref[...] = a * arnd


def _sc_body(syn_hbm, mask_hbm, rlist_hbm, adet_hbm, arnd_hbm, out_hbm,
             adet_v, arnd_v, r_v, syn_v, mask_v, out_v, sem):
    wid = lax.axis_index("s") * _NC + lax.axis_index("c")
    base = wid * _CHUNK

    copies = [
        pltpu.async_copy(adet_hbm, adet_v, sem),
        pltpu.async_copy(arnd_hbm, arnd_v, sem),
        pltpu.async_copy(rlist_hbm, r_v, sem),
        pltpu.async_copy(syn_hbm.at[:, pl.ds(base, _CHUNK)], syn_v, sem),
        pltpu.async_copy(mask_hbm.at[:, pl.ds(base, _CHUNK)], mask_v, sem),
    ]
    for c in copies:
        c.wait()

    r = r_v[...]
    d = lax.div(jnp.full((_L,), _SYN_LEN, jnp.int32), r)
    for g in range(_G):
        sl = pl.ds(g * _L, _L)
        p = lax.broadcasted_iota(jnp.int32, (_L,), 0) + (base + g * _L)
        q = lax.div(p, d)
        det_id = p - q * d
        rnd_id = jnp.minimum(q + 1, _MAX_ROUNDS)
        pe = (plsc.load_gather(adet_v, [det_id]) +
              plsc.load_gather(arnd_v, [rnd_id]))
        for b in range(_B):
            out_v[b, sl] = syn_v[b, sl] + mask_v[b, sl] * pe

    pltpu.sync_copy(out_v, out_hbm.at[:, pl.ds(base, _CHUNK)])


@jax.jit
def kernel(syn_bits, r_list, mask, det_emb_w, rnd_emb_w, proj_w, alpha):
    alpha2d = jnp.reshape(alpha, (1, 1)).astype(jnp.float32)

    adet, arnd = pl.pallas_call(
        _dots_tc,
        out_shape=(
            jax.ShapeDtypeStruct((1, _NUM_DETECTORS), jnp.float32),
            jax.ShapeDtypeStruct((1, _NRND), jnp.float32),
        ),
    )(det_emb_w, rnd_emb_w, proj_w, alpha2d)
    adet = jnp.reshape(adet, (_NUM_DETECTORS,))
    arnd = jnp.reshape(arnd, (_NRND,))

    mesh = plsc.VectorSubcoreMesh(core_axis_name="c", subcore_axis_name="s",
                                  num_cores=_NC, num_subcores=_NS)
    sc = pl.kernel(
        _sc_body,
        out_type=jax.ShapeDtypeStruct((_B, _SYN_LEN), jnp.float32),
        mesh=mesh,
        compiler_params=pltpu.CompilerParams(needs_layout_passes=False),
        scratch_types=[
            pltpu.VMEM((_NUM_DETECTORS,), jnp.float32),
            pltpu.VMEM((_NRND,), jnp.float32),
            pltpu.VMEM((_L,), jnp.int32),
            pltpu.VMEM((_B, _CHUNK), jnp.float32),
            pltpu.VMEM((_B, _CHUNK), jnp.float32),
            pltpu.VMEM((_B, _CHUNK), jnp.float32),
            pltpu.SemaphoreType.DMA,
        ],
    )
    return sc(syn_bits, mask, r_list, adet, arnd)
